# baseline (speedup 1.0000x reference)
import functools

import jax
import jax.numpy as jnp
from jax import lax
from jax.experimental import pallas as pl
from jax.experimental.pallas import tpu as pltpu
from jax.experimental.pallas import tpu_sc as plsc

D = 64
EPS = 1e-5
NC = 2
NS = 16
NW = NC * NS
K = 128


def _rsqrt(x):
    i = lax.bitcast_convert_type(x, jnp.int32)
    i = jnp.int32(0x5F3759DF) - lax.shift_right_arithmetic(i, jnp.int32(1))
    y = lax.bitcast_convert_type(i, jnp.float32)
    half = x * jnp.float32(0.5)
    for _ in range(3):
        y = y * (jnp.float32(1.5) - half * y * y)
    return y


def kernel(x, table, gamma, beta):
    Bt, L = x.shape
    B = Bt * L
    rows_per_w = B // NW
    chunks = rows_per_w // K
    x2d = x.reshape(B // 128, 128)

    mesh = plsc.VectorSubcoreMesh(core_axis_name="c", subcore_axis_name="s")

    @functools.partial(
        pl.kernel,
        mesh=mesh,
        compiler_params=pltpu.CompilerParams(
            needs_layout_passes=False, use_tc_tiling_on_sc=False
        ),
        out_type=jax.ShapeDtypeStruct((B, D), jnp.float32),
        scratch_types=[
            pltpu.VMEM((K // 128, 128), jnp.int32),
            pltpu.VMEM((K, D), jnp.float32),
            pltpu.VMEM((D,), jnp.float32),
            pltpu.VMEM((D,), jnp.float32),
            pltpu.SemaphoreType.DMA,
        ],
    )
    def sc_kernel(x_hbm, t_hbm, g_hbm, b_hbm, o_hbm, idx_v, rows_v, g_v, b_v, sem):
        wid = lax.axis_index("s") * NC + lax.axis_index("c")
        base = wid * rows_per_w
        pltpu.sync_copy(g_hbm, g_v)
        pltpu.sync_copy(b_hbm, b_v)
        gs = [g_v[pl.ds(16 * j, 16)] for j in range(4)]
        bs = [b_v[pl.ds(16 * j, 16)] for j in range(4)]

        def chunk_body(c, carry):
            row0 = base + c * K
            pltpu.sync_copy(x_hbm.at[pl.ds(row0 // 128, K // 128)], idx_v)
            pltpu.async_copy(t_hbm.at[idx_v.at[0]], rows_v, sem).wait()

            def row_body(i, carry2):
                vs = [rows_v[i, pl.ds(16 * j, 16)] for j in range(4)]
                total = jnp.sum(vs[0] + vs[1] + vs[2] + vs[3])
                mean = total * jnp.float32(1.0 / D)
                ts = [v - mean for v in vs]
                q = ts[0] * ts[0] + ts[1] * ts[1] + ts[2] * ts[2] + ts[3] * ts[3]
                var = jnp.sum(q) * jnp.float32(1.0 / D)
                rstd = _rsqrt(var + jnp.float32(EPS))
                for j in range(4):
                    rows_v[i, pl.ds(16 * j, 16)] = ts[j] * rstd * gs[j] + bs[j]
                return carry2

            lax.fori_loop(0, K, row_body, 0)
            pltpu.sync_copy(rows_v, o_hbm.at[pl.ds(row0, K)])
            return carry

        lax.fori_loop(0, chunks, chunk_body, 0)

    out = sc_kernel(x2d, table, gamma, beta)
    return out.reshape(Bt, L, D)


# speedup vs baseline: 1.7538x; 1.7538x over previous
import functools

import jax
import jax.numpy as jnp
from jax import lax
from jax.experimental import pallas as pl
from jax.experimental.pallas import tpu as pltpu
from jax.experimental.pallas import tpu_sc as plsc

D = 64
EPS = 1e-5
NC = 2
NS = 16
NW = NC * NS
K = 128


def _rsqrt(x):
    i = lax.bitcast_convert_type(x, jnp.int32)
    i = jnp.int32(0x5F3759DF) - lax.shift_right_arithmetic(i, jnp.int32(1))
    y = lax.bitcast_convert_type(i, jnp.float32)
    half = x * jnp.float32(0.5)
    for _ in range(3):
        y = y * (jnp.float32(1.5) - half * y * y)
    return y


def kernel(x, table, gamma, beta):
    Bt, L = x.shape
    B = Bt * L
    rows_per_w = B // NW
    chunks = rows_per_w // K
    x2d = x.reshape(B // 128, 128)

    mesh = plsc.VectorSubcoreMesh(core_axis_name="c", subcore_axis_name="s")

    @functools.partial(
        pl.kernel,
        mesh=mesh,
        compiler_params=pltpu.CompilerParams(
            needs_layout_passes=False, use_tc_tiling_on_sc=False
        ),
        out_type=jax.ShapeDtypeStruct((B, D), jnp.float32),
        scratch_types=[
            pltpu.VMEM((K // 128, 128), jnp.int32),
            pltpu.VMEM((K, D), jnp.float32),
            pltpu.VMEM((D,), jnp.float32),
            pltpu.VMEM((D,), jnp.float32),
            pltpu.SemaphoreType.DMA,
        ],
    )
    def sc_kernel(x_hbm, t_hbm, g_hbm, b_hbm, o_hbm, idx_v, rows_v, g_v, b_v, sem):
        wid = lax.axis_index("s") * NC + lax.axis_index("c")
        base = wid * rows_per_w
        pltpu.sync_copy(g_hbm, g_v)
        pltpu.sync_copy(b_hbm, b_v)
        gs = [g_v[pl.ds(16 * j, 16)] for j in range(4)]
        bs = [b_v[pl.ds(16 * j, 16)] for j in range(4)]

        def chunk_body(c, carry):
            row0 = base + c * K
            pltpu.sync_copy(x_hbm.at[pl.ds(row0 // 128, K // 128)], idx_v)
            pltpu.async_copy(t_hbm.at[idx_v.at[0]], rows_v, sem).wait()

            def row_body(i, carry2):
                vs = [rows_v[i, pl.ds(16 * j, 16)] for j in range(4)]
                total = jnp.sum(vs[0] + vs[1] + vs[2] + vs[3])
                mean = total * jnp.float32(1.0 / D)
                ts = [v - mean for v in vs]
                q = ts[0] * ts[0] + ts[1] * ts[1] + ts[2] * ts[2] + ts[3] * ts[3]
                var = jnp.sum(q) * jnp.float32(1.0 / D)
                rstd = _rsqrt(var + jnp.float32(EPS))
                for j in range(4):
                    rows_v[i, pl.ds(16 * j, 16)] = ts[j] * rstd * gs[j] + bs[j]
                return carry2

            lax.fori_loop(0, K, row_body, 0, unroll=8)
            pltpu.sync_copy(rows_v, o_hbm.at[pl.ds(row0, K)])
            return carry

        lax.fori_loop(0, chunks, chunk_body, 0)

    out = sc_kernel(x2d, table, gamma, beta)
    return out.reshape(Bt, L, D)


# speedup vs baseline: 2.1633x; 1.2335x over previous
import functools

import jax
import jax.numpy as jnp
from jax import lax
from jax.experimental import pallas as pl
from jax.experimental.pallas import tpu as pltpu
from jax.experimental.pallas import tpu_sc as plsc

D = 64
EPS = 1e-5
NC = 2
NS = 16
NW = NC * NS
G = 128
NBUF = 4


def _rsqrt(x):
    i = lax.bitcast_convert_type(x, jnp.int32)
    i = jnp.int32(0x5F3759DF) - lax.shift_right_arithmetic(i, jnp.int32(1))
    y = lax.bitcast_convert_type(i, jnp.float32)
    half = x * jnp.float32(0.5)
    for _ in range(3):
        y = y * (jnp.float32(1.5) - half * y * y)
    return y


def kernel(x, table, gamma, beta):
    Bt, L = x.shape
    B = Bt * L
    rows_per_w = B // NW
    N = rows_per_w // G
    x2d = x.reshape(B // G, G)

    mesh = plsc.VectorSubcoreMesh(core_axis_name="c", subcore_axis_name="s")

    @functools.partial(
        pl.kernel,
        mesh=mesh,
        compiler_params=pltpu.CompilerParams(
            needs_layout_passes=False, use_tc_tiling_on_sc=False
        ),
        out_type=jax.ShapeDtypeStruct((B, D), jnp.float32),
        scratch_types=[
            pltpu.VMEM((N, G), jnp.int32),
            pltpu.VMEM((G, D), jnp.float32),
            pltpu.VMEM((G, D), jnp.float32),
            pltpu.VMEM((G, D), jnp.float32),
            pltpu.VMEM((G, D), jnp.float32),
            pltpu.VMEM((D,), jnp.float32),
            pltpu.VMEM((D,), jnp.float32),
            pltpu.SemaphoreType.DMA,
            pltpu.SemaphoreType.DMA,
            pltpu.SemaphoreType.DMA,
            pltpu.SemaphoreType.DMA,
            pltpu.SemaphoreType.DMA,
            pltpu.SemaphoreType.DMA,
            pltpu.SemaphoreType.DMA,
            pltpu.SemaphoreType.DMA,
        ],
    )
    def sc_kernel(x_hbm, t_hbm, g_hbm, b_hbm, o_hbm,
                  idx_all, r0, r1, r2, r3, g_v, b_v,
                  gs0, gs1, gs2, gs3, ss0, ss1, ss2, ss3):
        wid = lax.axis_index("s") * NC + lax.axis_index("c")
        base = wid * rows_per_w
        rbufs = [r0, r1, r2, r3]
        gsems = [gs0, gs1, gs2, gs3]
        ssems = [ss0, ss1, ss2, ss3]

        pltpu.sync_copy(g_hbm, g_v)
        pltpu.sync_copy(b_hbm, b_v)
        pltpu.sync_copy(x_hbm.at[pl.ds(wid * N, N)], idx_all)
        gs = [g_v[pl.ds(16 * j, 16)] for j in range(4)]
        bs = [b_v[pl.ds(16 * j, 16)] for j in range(4)]

        def gstart(c, b):
            pltpu.async_copy(t_hbm.at[idx_all.at[c]], rbufs[b], gsems[b])

        def gwait(c, b):
            pltpu.make_async_copy(
                t_hbm.at[idx_all.at[c]], rbufs[b], gsems[b]
            ).wait()

        def ostart(c, b):
            pltpu.async_copy(
                rbufs[b], o_hbm.at[pl.ds(base + c * G, G)], ssems[b]
            )

        def owait(c, b):
            pltpu.make_async_copy(
                rbufs[b], o_hbm.at[pl.ds(base + c * G, G)], ssems[b]
            ).wait()

        def compute(b):
            rows_v = rbufs[b]

            def row_body(i, carry):
                vs = [rows_v[i, pl.ds(16 * j, 16)] for j in range(4)]
                total = jnp.sum(vs[0] + vs[1] + vs[2] + vs[3])
                mean = total * jnp.float32(1.0 / D)
                ts = [v - mean for v in vs]
                q = ts[0] * ts[0] + ts[1] * ts[1] + ts[2] * ts[2] + ts[3] * ts[3]
                var = jnp.sum(q) * jnp.float32(1.0 / D)
                rstd = _rsqrt(var + jnp.float32(EPS))
                for j in range(4):
                    rows_v[i, pl.ds(16 * j, 16)] = ts[j] * rstd * gs[j] + bs[j]
                return carry

            lax.fori_loop(0, G, row_body, 0, unroll=8)

        gstart(0, 0)
        gstart(1, 1)
        gstart(2, 2)

        gwait(0, 0)
        compute(0)
        ostart(0, 0)
        gstart(3, 3)
        for b in range(1, NBUF):
            gwait(b, b)
            compute(b)
            ostart(b, b)
            owait(b - 1, b - 1)
            gstart(b + 3, (b + 3) % NBUF)

        def group_body(g, carry):
            c0 = g * NBUF
            for b in range(NBUF):
                c = c0 + b
                gwait(c, b)
                compute(b)
                ostart(c, b)
                owait(c - 1, (b + 3) % NBUF)
                gstart(c + 3, (b + 3) % NBUF)
            return carry

        lax.fori_loop(1, N // NBUF - 1, group_body, 0)

        cL = N - NBUF
        gwait(cL, 0)
        compute(0)
        ostart(cL, 0)
        owait(cL - 1, 3)
        gstart(N - 1, 3)
        for b in range(1, NBUF):
            c = cL + b
            gwait(c, b)
            compute(b)
            ostart(c, b)
            owait(c - 1, b - 1)
        owait(N - 1, NBUF - 1)

    out = sc_kernel(x2d, table, gamma, beta)
    return out.reshape(Bt, L, D)
